# Initial kernel scaffold; baseline (speedup 1.0000x reference)
import jax
import jax.numpy as jnp
from jax.experimental import pallas as pl


def kernel(x, edge_index, Wl1, bl1, Wr1, g1, b1, Wl2, bl2, Wr2, g2, b2, Wl3, bl3, Wr3, g3, b3, W4, b4, g4, b4n, W5, b5):
    raise NotImplementedError("write your pallas kernel here")


# speedup vs baseline: 3.7979x; 3.7979x over previous
import jax
import jax.numpy as jnp
from jax import lax
from jax.experimental import pallas as pl
from jax.experimental.pallas import tpu as pltpu
from jax.experimental.pallas import tpu_sc as plsc

N = 10000
E = 320000
D = 128
L = 16
NC, NS = 2, 16
NW = NC * NS

NP2 = 10240
HALF = NP2 // NC
NTRASH = 16
HT = HALF + NTRASH
E2 = 327680
EROWS = E2 // 128
C_MEAN = 1024
EPT = E2 // NS
ITERS_MEAN = EPT // C_MEAN
R_OWN = NP2 // NW

_mesh = plsc.VectorSubcoreMesh(core_axis_name="c", subcore_axis_name="s")


def _mean_body(feat, src2d, dst2d, zacc, zcnt, acc_out, cnt_out,
               src_v, dst_v, dadj_v, rows_v, ones_v, acc_sh, cnt_sh, sem):
    c = lax.axis_index("c")
    s = lax.axis_index("s")
    base = c * HALF
    iota16 = lax.iota(jnp.int32, L)
    one16 = jnp.ones((L,), jnp.float32)
    trash16 = HALF + iota16

    pltpu.sync_copy(zacc, acc_sh.at[pl.ds(s * (HT // NS), HT // NS)])
    pltpu.sync_copy(zcnt, cnt_sh.at[pl.ds(s * (HT // NS), HT // NS)])

    def init_ones(r, carry):
        ones_v[r, :] = one16
        return carry
    lax.fori_loop(0, 128, init_ones, 0)
    plsc.subcore_barrier()

    row_base = s * (EPT // 128)

    def edge_iter(i, carry):
        roff = row_base + i * (C_MEAN // 128)
        pltpu.sync_copy(src2d.at[pl.ds(roff, C_MEAN // 128)], src_v)
        pltpu.sync_copy(dst2d.at[pl.ds(roff, C_MEAN // 128)], dst_v)
        for j in range(C_MEAN // 128):
            for o in range(8):
                d16 = dst_v[j, pl.ds(o * L, L)]
                dl = d16 - base
                msk = (dl >= 0) & (dl < HALF)
                dadj_v[j, pl.ds(o * L, L)] = jnp.where(msk, dl, trash16)
        for j in range(C_MEAN // 128):
            pltpu.async_copy(feat.at[src_v.at[j]], rows_v, sem).wait()
            pltpu.sync_copy(rows_v, acc_sh.at[dadj_v.at[j]], add=True)
            pltpu.sync_copy(ones_v, cnt_sh.at[dadj_v.at[j]], add=True)
        return carry

    lax.fori_loop(0, ITERS_MEAN, edge_iter, 0)
    plsc.subcore_barrier()

    out_off = s * (HALF // NS)
    pltpu.sync_copy(acc_sh.at[pl.ds(out_off, HALF // NS)],
                    acc_out.at[pl.ds(c * HALF + out_off, HALF // NS)])
    pltpu.sync_copy(cnt_sh.at[pl.ds(out_off, HALF // NS)],
                    cnt_out.at[pl.ds(c * HALF + out_off, HALF // NS)])


_mean_call = pl.kernel(
    _mean_body,
    out_type=[jax.ShapeDtypeStruct((NP2, D), jnp.float32),
              jax.ShapeDtypeStruct((NP2, L), jnp.float32)],
    mesh=_mesh,
    scratch_types=[
        pltpu.VMEM((C_MEAN // 128, 128), jnp.int32),
        pltpu.VMEM((C_MEAN // 128, 128), jnp.int32),
        pltpu.VMEM((C_MEAN // 128, 128), jnp.int32),
        pltpu.VMEM((128, D), jnp.float32),
        pltpu.VMEM((128, L), jnp.float32),
        pltpu.VMEM_SHARED((HT, D), jnp.float32),
        pltpu.VMEM_SHARED((HT, L), jnp.float32),
        pltpu.SemaphoreType.DMA,
    ],
)


def _max_body(feat, ssrc2d, sdst2d, cuts, zacc, out,
              cut_v, src_v, dst_v, rows_v, acc_v, sem):
    c = lax.axis_index("c")
    s = lax.axis_index("s")
    wid = c * NS + s
    lo = wid * R_OWN

    pltpu.sync_copy(zacc, acc_v)

    pltpu.sync_copy(cuts, cut_v)
    c16 = cut_v[wid, pl.ds(0, L)]
    start = c16[0]
    end = c16[1]
    blk0 = start // 1024
    nblk = (end + 1023) // 1024 - blk0

    def blk_iter(jb, carry):
        row = (blk0 + jb) * 8
        pltpu.sync_copy(sdst2d.at[pl.ds(row, 8)], dst_v)
        pltpu.sync_copy(ssrc2d.at[pl.ds(row, 8)], src_v)

        for jr in range(8):
            pltpu.async_copy(feat.at[src_v.at[jr]], rows_v, sem).wait()

            def grp(o, carry2):
                d16 = dst_v[jr, pl.ds(o * L, L)]
                dl16 = d16 - lo
                for l in range(L):
                    d = dl16[l]

                    @pl.when((d >= 0) & (d < R_OWN))
                    def _():
                        e = o * L + l
                        for cc in range(D // L):
                            a = acc_v[d, pl.ds(cc * L, L)]
                            m = rows_v[e, pl.ds(cc * L, L)]
                            acc_v[d, pl.ds(cc * L, L)] = jnp.maximum(a, m)
                return carry2

            lax.fori_loop(0, 8, grp, 0)
        return carry

    lax.fori_loop(0, nblk, blk_iter, 0)
    pltpu.sync_copy(acc_v, out.at[pl.ds(wid * R_OWN, R_OWN)])


_max_call = pl.kernel(
    _max_body,
    out_type=jax.ShapeDtypeStruct((NP2, D), jnp.float32),
    mesh=_mesh,
    scratch_types=[
        pltpu.VMEM((NW, L), jnp.int32),
        pltpu.VMEM((8, 128), jnp.int32),
        pltpu.VMEM((8, 128), jnp.int32),
        pltpu.VMEM((128, D), jnp.float32),
        pltpu.VMEM((R_OWN, D), jnp.float32),
        pltpu.SemaphoreType.DMA,
    ],
)


BLK = 512
GRID = 20


def _ln_relu(h, g, b):
    m = jnp.mean(h, axis=-1, keepdims=True)
    v = jnp.mean((h - m) ** 2, axis=-1, keepdims=True)
    return jnp.maximum((h - m) * lax.rsqrt(v + 1e-5) * g + b, 0.0)


def _dot(a, b):
    return jnp.dot(a, b, preferred_element_type=jnp.float32)


def _stage_a_body(acc, cnt, xr, wl, bl, wr, g, b, out):
    agg = acc[...] / jnp.maximum(cnt[:, 0:1], 1.0)
    h = _dot(agg, wl[...]) + bl[...] + _dot(xr[...], wr[...])
    out[...] = _ln_relu(h, g[...], b[...])


def _stage_b_body(agg2, h1, wl2, bl2, wr2, g2, b2, wl3, wr3, y3, z3):
    h = _dot(agg2[...], wl2[...]) + bl2[...] + _dot(h1[...], wr2[...])
    h2 = _ln_relu(h, g2[...], b2[...])
    y3[...] = _dot(h2, wl3[...])
    z3[...] = _dot(h2, wr3[...])


def _stage_c_body(acc, cnt, z3, bl3, g3, b3, w4, b4, g4, b4n, w5, b5, out):
    agg = acc[...] / jnp.maximum(cnt[:, 0:1], 1.0)
    h3 = _ln_relu(agg + bl3[...] + z3[...], g3[...], b3[...])
    h4 = _ln_relu(_dot(h3, w4[...]) + b4[...], g4[...], b4n[...])
    out[...] = _dot(h4, w5[...]) + b5[...]


def _row_spec(width):
    return pl.BlockSpec((BLK, width), lambda i: (i, 0))


def _full_spec(a, b):
    return pl.BlockSpec((a, b), lambda i: (0, 0))


def _stage_a(acc, cnt, x, wlT, bl, wrT, g, b):
    return pl.pallas_call(
        _stage_a_body,
        grid=(GRID,),
        in_specs=[_row_spec(D), _row_spec(L), _row_spec(D), _full_spec(D, D),
                  _full_spec(1, D), _full_spec(D, D), _full_spec(1, D),
                  _full_spec(1, D)],
        out_specs=_row_spec(D),
        out_shape=jax.ShapeDtypeStruct((N, D), jnp.float32),
    )(acc, cnt, x, wlT, bl, wrT, g, b)


def _stage_b(agg2, h1, wl2T, bl2, wr2T, g2, b2, wl3T, wr3T):
    return pl.pallas_call(
        _stage_b_body,
        grid=(GRID,),
        in_specs=[_row_spec(D), _row_spec(D), _full_spec(D, 256),
                  _full_spec(1, 256), _full_spec(D, 256), _full_spec(1, 256),
                  _full_spec(1, 256), _full_spec(256, D), _full_spec(256, D)],
        out_specs=[_row_spec(D), _row_spec(D)],
        out_shape=[jax.ShapeDtypeStruct((N, D), jnp.float32),
                   jax.ShapeDtypeStruct((N, D), jnp.float32)],
    )(agg2, h1, wl2T, bl2, wr2T, g2, b2, wl3T, wr3T)


def _stage_c(acc, cnt, z3, bl3, g3, b3, w4T, b4, g4, b4n, w5T, b5):
    return pl.pallas_call(
        _stage_c_body,
        grid=(GRID,),
        in_specs=[_row_spec(D), _row_spec(L), _row_spec(D), _full_spec(1, D),
                  _full_spec(1, D), _full_spec(1, D), _full_spec(D, 64),
                  _full_spec(1, 64), _full_spec(1, 64), _full_spec(1, 64),
                  _full_spec(64, 64), _full_spec(1, 64)],
        out_specs=_row_spec(64),
        out_shape=jax.ShapeDtypeStruct((N, 64), jnp.float32),
    )(acc, cnt, z3, bl3, g3, b3, w4T, b4, g4, b4n, w5T, b5)


def kernel(x, edge_index, Wl1, bl1, Wr1, g1, b1, Wl2, bl2, Wr2, g2, b2,
           Wl3, bl3, Wr3, g3, b3, W4, b4, g4, b4n, W5, b5):
    src = edge_index[0]
    dst = edge_index[1]
    pad = E2 - E
    src_p = jnp.concatenate([src, jnp.arange(pad, dtype=jnp.int32) % N])
    dst_p = jnp.concatenate(
        [dst, N + (jnp.arange(pad, dtype=jnp.int32) % NTRASH)])
    src2d = src_p.reshape(EROWS, 128)
    dst2d = dst_p.reshape(EROWS, 128)

    sdst, ssrc = lax.sort([dst_p, src_p], num_keys=1)
    cuts = jnp.searchsorted(
        sdst, (jnp.arange(NW + 1) * R_OWN).astype(jnp.int32)
    ).astype(jnp.int32)
    cuts_info = jnp.zeros((NW, L), jnp.int32)
    cuts_info = cuts_info.at[:, 0].set(cuts[:NW]).at[:, 1].set(cuts[1:])
    ssrc2d = ssrc.reshape(EROWS, 128)
    sdst2d = sdst.reshape(EROWS, 128)

    zacc = jnp.zeros((HT // NS, D), jnp.float32)
    zcnt = jnp.zeros((HT // NS, L), jnp.float32)
    zaccm = jnp.zeros((R_OWN, D), jnp.float32)

    r1 = lambda a: a.reshape(1, -1)

    acc1, cnt1 = _mean_call(x, src2d, dst2d, zacc, zcnt)
    h1 = _stage_a(acc1, cnt1, x, Wl1.T, r1(bl1), Wr1.T, r1(g1), r1(b1))

    agg2 = _max_call(h1, ssrc2d, sdst2d, cuts_info, zaccm)
    y3, z3 = _stage_b(agg2, h1, Wl2.T, r1(bl2), Wr2.T, r1(g2), r1(b2),
                      Wl3.T, Wr3.T)

    acc3, cnt3 = _mean_call(y3, src2d, dst2d, zacc, zcnt)
    out = _stage_c(acc3, cnt3, z3, r1(bl3), r1(g3), r1(b3), W4.T,
                   r1(b4), r1(g4), r1(b4n), W5.T, r1(b5))
    return out
